# Initial kernel scaffold; baseline (speedup 1.0000x reference)
import jax
import jax.numpy as jnp
from jax.experimental import pallas as pl


def kernel(users, items, user_table, item_table):
    raise NotImplementedError("write your pallas kernel here")


# speedup vs baseline: 1.1019x; 1.1019x over previous
import functools

import jax
import jax.numpy as jnp
from jax import lax
from jax.experimental import pallas as pl
from jax.experimental.pallas import tpu as pltpu, tpu_sc as plsc

NC, NS, L = 2, 16, 16
NW = NC * NS

B = 16384
D = 128
BPW = B // NW
C = 128
NCHUNK = BPW // C
G = C // L


def _mf_body(users, items, ut, it, out, uidx, iidx, urows, irows, outbuf,
             sem_u, sem_i):
    wid = lax.axis_index("s") * NC + lax.axis_index("c")
    base = wid * BPW
    pltpu.sync_copy(users.at[pl.ds(base, BPW)], uidx)
    pltpu.sync_copy(items.at[pl.ds(base, BPW)], iidx)
    lane = lax.iota(jnp.int32, 16)

    @pl.loop(0, NCHUNK)
    def _chunk(c):
        cu = pltpu.async_copy(ut.at[uidx.at[pl.ds(c * C, C)]], urows, sem_u)
        ci = pltpu.async_copy(it.at[iidx.at[pl.ds(c * C, C)]], irows, sem_i)
        cu.wait()
        ci.wait()

        @pl.loop(0, G)
        def _grp(g):
            res = jnp.zeros((L,), jnp.float32)
            for rr in range(L):
                r = g * L + rr
                acc = jnp.zeros((L,), jnp.float32)
                for k in range(D // L):
                    acc = acc + urows[r, pl.ds(k * L, L)] * irows[r, pl.ds(k * L, L)]
                s = jnp.sum(acc)
                res = jnp.where(lane == rr, s, res)
            outbuf[pl.ds(c * C + g * L, L)] = res * (1.0 / D)

    pltpu.sync_copy(outbuf, out.at[pl.ds(base, BPW)])


@jax.jit
def kernel(users, items, user_table, item_table):
    mesh = plsc.VectorSubcoreMesh(core_axis_name="c", subcore_axis_name="s")
    f = pl.kernel(
        _mf_body,
        out_type=jax.ShapeDtypeStruct((B,), jnp.float32),
        mesh=mesh,
        compiler_params=pltpu.CompilerParams(needs_layout_passes=False),
        scratch_types=[
            pltpu.VMEM((BPW,), jnp.int32),
            pltpu.VMEM((BPW,), jnp.int32),
            pltpu.VMEM((C, D), jnp.float32),
            pltpu.VMEM((C, D), jnp.float32),
            pltpu.VMEM((BPW,), jnp.float32),
            pltpu.SemaphoreType.DMA,
            pltpu.SemaphoreType.DMA,
        ],
    )
    return f(users.astype(jnp.int32), items.astype(jnp.int32),
             user_table, item_table)
